# baseline (speedup 1.0000x reference)
import functools

import jax
import jax.numpy as jnp
from jax.experimental import pallas as pl
from jax.experimental.pallas import tpu as pltpu
from jax.experimental.pallas import tpu_sc as plsc

E = 8
K = 2
D = 768
H = 1536
S = 2048
T = 256
NT = 24
P = NT * T
CHUNK = 256


def _routing_body(x_ref, wg_ref, bg_ref, pos0_ref, pos1_ref, w0_ref, w1_ref,
                  te_ref, act_ref, first_ref, nxt_ref, do_ref, bufp_ref,
                  nxt2_ref, do2_ref):
    x = x_ref[...]
    logits = jnp.dot(x, wg_ref[...], preferred_element_type=jnp.float32)
    logits = logits + bg_ref[...]
    eidx = jax.lax.broadcasted_iota(jnp.int32, (S, E), 1)
    m0 = jnp.max(logits, axis=1, keepdims=True)
    a0 = jnp.argmax(logits, axis=1).reshape(-1, 1)
    masked = jnp.where(eidx == a0, -jnp.inf, logits)
    m1 = jnp.max(masked, axis=1, keepdims=True)
    a1 = jnp.argmax(masked, axis=1).reshape(-1, 1)
    w0 = 1.0 / (1.0 + jnp.exp(m1 - m0))
    w1 = 1.0 - w0
    w0_ref[...] = jnp.broadcast_to(w0, (S, 128))
    w1_ref[...] = jnp.broadcast_to(w1, (S, 128))

    oh0 = (eidx == a0).astype(jnp.float32)
    oh1 = (eidx == a1).astype(jnp.float32)
    ohsum = oh0 + oh1

    r_i = jax.lax.broadcasted_iota(jnp.int32, (CHUNK, CHUNK), 0)
    c_i = jax.lax.broadcasted_iota(jnp.int32, (CHUNK, CHUNK), 1)
    Lt = (c_i < r_i).astype(jnp.float32)
    carry = jnp.zeros((1, E), jnp.float32)
    excs = []
    for i in range(S // CHUNK):
        blk = ohsum[i * CHUNK:(i + 1) * CHUNK, :]
        excs.append(jnp.dot(Lt, blk, preferred_element_type=jnp.float32) + carry)
        carry = carry + jnp.sum(blk, axis=0, keepdims=True)
    exc = jnp.concatenate(excs, axis=0)
    counts = carry

    ci = counts.astype(jnp.int32)
    pc = ((ci + (T - 1)) // T) * T
    e_r = jax.lax.broadcasted_iota(jnp.int32, (E, E), 0)
    e_c = jax.lax.broadcasted_iota(jnp.int32, (E, E), 1)
    base = jnp.sum(jnp.where(e_c < e_r, jnp.broadcast_to(pc, (E, E)), 0),
                   axis=1).reshape(1, E)
    cc = base + pc

    basef = base.astype(jnp.float32)
    pos0 = jnp.sum(oh0 * (basef + exc), axis=1, keepdims=True)
    pos1 = jnp.sum(oh1 * (basef + exc), axis=1, keepdims=True)
    pos0_ref[...] = pos0.astype(jnp.int32)
    pos1_ref[...] = pos1.astype(jnp.int32)

    t_i = jax.lax.broadcasted_iota(jnp.int32, (NT, E), 0) * T
    te = jnp.sum((t_i >= jnp.broadcast_to(cc, (NT, E))).astype(jnp.int32),
                 axis=1, keepdims=True)
    act = (te < E).astype(jnp.int32)
    act_ref[...] = act
    te_ref[...] = jnp.minimum(te, E - 1)

    te_prev = jnp.concatenate([jnp.full((1, 1), -1, jnp.int32), te[:-1]], axis=0)
    first = ((te != te_prev) & (act == 1)).astype(jnp.int32)
    tt_r = jax.lax.broadcasted_iota(jnp.int32, (NT, NT), 0)
    tt_c = jax.lax.broadcasted_iota(jnp.int32, (NT, NT), 1)
    g = jnp.sum(jnp.where(tt_c <= tt_r,
                          jnp.broadcast_to(first.reshape(1, NT), (NT, NT)), 0),
                axis=1, keepdims=True) - 1
    bufp_ref[...] = jnp.maximum(g, 0) % 3
    e_ids = jax.lax.broadcasted_iota(jnp.int32, (NT, E), 1)
    has = jnp.broadcast_to(ci, (NT, E)) > 0
    cand = jnp.where((e_ids > jnp.minimum(te, E - 1)) & has, e_ids, 99)
    nxt = jnp.min(cand, axis=1, keepdims=True)
    cand2 = jnp.where((e_ids > nxt) & has, e_ids, 99)
    nxt2 = jnp.min(cand2, axis=1, keepdims=True)
    do = ((nxt < E) & (first == 1)).astype(jnp.int32)
    do2 = ((nxt2 < E) & (first == 1)).astype(jnp.int32)
    first_ref[...] = first
    do_ref[...] = do
    nxt_ref[...] = jnp.where(nxt < E, nxt, 0)
    do2_ref[...] = do2
    nxt2_ref[...] = jnp.where(nxt2 < E, nxt2, 0)


@functools.partial(jax.jit)
def _routing(x2, Wg, bg):
    return pl.pallas_call(
        _routing_body,
        out_shape=[
            jax.ShapeDtypeStruct((S, 1), jnp.int32),
            jax.ShapeDtypeStruct((S, 1), jnp.int32),
            jax.ShapeDtypeStruct((S, 128), jnp.float32),
            jax.ShapeDtypeStruct((S, 128), jnp.float32),
            jax.ShapeDtypeStruct((NT, 1), jnp.int32),
            jax.ShapeDtypeStruct((NT, 1), jnp.int32),
            jax.ShapeDtypeStruct((NT, 1), jnp.int32),
            jax.ShapeDtypeStruct((NT, 1), jnp.int32),
            jax.ShapeDtypeStruct((NT, 1), jnp.int32),
            jax.ShapeDtypeStruct((NT, 1), jnp.int32),
            jax.ShapeDtypeStruct((NT, 1), jnp.int32),
            jax.ShapeDtypeStruct((NT, 1), jnp.int32),
        ],
    )(x2, Wg, bg)


def _ffn_body(te_ref, act_ref, first_ref, nxt_ref, do_ref, bufp_ref,
              nxt2_ref, do2_ref,
              xs_ref, rw_ref, wa_any, w1_any, w2_any, ba_ref, b1_ref, b2_ref,
              y_ref, wab, w1b, w2b, sa, s1, s2):
    t = pl.program_id(0)
    p = bufp_ref[t]

    def _w_copy(e, slot):
        return (
            pltpu.make_async_copy(wa_any.at[e], wab.at[slot], sa.at[slot]),
            pltpu.make_async_copy(w1_any.at[e], w1b.at[slot], s1.at[slot]),
            pltpu.make_async_copy(w2_any.at[e], w2b.at[slot], s2.at[slot]),
        )

    @pl.when(t == 0)
    def _prime():
        for c in _w_copy(te_ref[0], 0):
            c.start()

        @pl.when(do_ref[0] == 1)
        def _prime2():
            for c in _w_copy(nxt_ref[0], 1):
                c.start()

    @pl.when(first_ref[t] == 1)
    def _stream():
        for c in _w_copy(te_ref[t], p):
            c.wait()

        @pl.when(do2_ref[t] == 1)
        def _prefetch():
            for c in _w_copy(nxt2_ref[t], (p + 2) % 3):
                c.start()

    @pl.when(act_ref[t] == 1)
    def _go():
        x = xs_ref[...]
        e = te_ref[t]
        a = jnp.dot(x, wab[p], preferred_element_type=jnp.float32) + ba_ref[e]
        a = a * jax.nn.sigmoid(a)
        f1 = jnp.dot(x, w1b[p], preferred_element_type=jnp.float32) + b1_ref[e]
        h = a * f1
        o = jnp.dot(h, w2b[p], preferred_element_type=jnp.float32) + b2_ref[e]
        y_ref[...] = o * rw_ref[...][:, 0:1]

    @pl.when(act_ref[t] == 0)
    def _skip():
        y_ref[...] = jnp.zeros_like(y_ref)


@functools.partial(jax.jit)
def _ffn(xs, rw, te, act, first, nxt, do, bufp, nxt2, do2,
         Wa, ba, W1, b1, W2, b2):
    grid_spec = pltpu.PrefetchScalarGridSpec(
        num_scalar_prefetch=8,
        grid=(NT,),
        in_specs=[
            pl.BlockSpec((T, D), lambda t, *_: (t, 0)),
            pl.BlockSpec((T, 128), lambda t, *_: (t, 0)),
            pl.BlockSpec(memory_space=pl.ANY),
            pl.BlockSpec(memory_space=pl.ANY),
            pl.BlockSpec(memory_space=pl.ANY),
            pl.BlockSpec((E, 1, H), lambda t, *_: (0, 0, 0)),
            pl.BlockSpec((E, 1, H), lambda t, *_: (0, 0, 0)),
            pl.BlockSpec((E, 1, D), lambda t, *_: (0, 0, 0)),
        ],
        out_specs=pl.BlockSpec((T, D), lambda t, *_: (t, 0)),
        scratch_shapes=[
            pltpu.VMEM((3, D, H), jnp.float32),
            pltpu.VMEM((3, D, H), jnp.float32),
            pltpu.VMEM((3, H, D), jnp.float32),
            pltpu.SemaphoreType.DMA((3,)),
            pltpu.SemaphoreType.DMA((3,)),
            pltpu.SemaphoreType.DMA((3,)),
        ],
    )
    return pl.pallas_call(
        _ffn_body,
        grid_spec=grid_spec,
        out_shape=jax.ShapeDtypeStruct((P, D), jnp.float32),
        compiler_params=pltpu.CompilerParams(
            dimension_semantics=("arbitrary",),
        ),
    )(te, act, first, nxt, do, bufp, nxt2, do2, xs, rw, Wa, W1, W2,
      ba.reshape(E, 1, H), b1.reshape(E, 1, H), b2.reshape(E, 1, D))


_SC_NC = 2
_SC_NS = 16
_NW = _SC_NC * _SC_NS
_TPW = S // _NW


@functools.cache
def _sc_kernels():
    mesh = plsc.VectorSubcoreMesh(core_axis_name="c", subcore_axis_name="s",
                                  num_cores=_SC_NC, num_subcores=_SC_NS)

    @functools.partial(
        pl.kernel,
        out_type=[
            jax.ShapeDtypeStruct((P, D), jnp.float32),
            jax.ShapeDtypeStruct((P, 128), jnp.float32),
        ],
        mesh=mesh,
        scratch_types=[
            pltpu.VMEM((_TPW,), jnp.int32),
            pltpu.VMEM((_TPW,), jnp.int32),
            pltpu.VMEM((_TPW, D), jnp.float32),
            pltpu.VMEM((_TPW, 128), jnp.float32),
            pltpu.VMEM((_TPW, 128), jnp.float32),
            pltpu.SemaphoreType.DMA,
            pltpu.SemaphoreType.DMA,
            pltpu.SemaphoreType.DMA,
            pltpu.SemaphoreType.DMA,
        ],
    )
    def _sc_dispatch(x_hbm, pos0_hbm, pos1_hbm, w0_hbm, w1_hbm, xs_hbm, rw_hbm,
                     idx0_v, idx1_v, rows_v, w0_v, w1_v, s0, s1, s2, s3):
        wid = jax.lax.axis_index("s") * _SC_NC + jax.lax.axis_index("c")
        base = wid * _TPW
        pltpu.sync_copy(pos0_hbm.at[wid], idx0_v)
        pltpu.sync_copy(pos1_hbm.at[wid], idx1_v)
        pltpu.sync_copy(x_hbm.at[pl.ds(base, _TPW)], rows_v)
        pltpu.sync_copy(w0_hbm.at[pl.ds(base, _TPW)], w0_v)
        pltpu.sync_copy(w1_hbm.at[pl.ds(base, _TPW)], w1_v)
        c0 = pltpu.async_copy(rows_v, xs_hbm.at[idx0_v], s0)
        c1 = pltpu.async_copy(rows_v, xs_hbm.at[idx1_v], s1)
        c2 = pltpu.async_copy(w0_v, rw_hbm.at[idx0_v], s2)
        c3 = pltpu.async_copy(w1_v, rw_hbm.at[idx1_v], s3)
        c0.wait()
        c1.wait()
        c2.wait()
        c3.wait()


    @functools.partial(
        pl.kernel,
        out_type=jax.ShapeDtypeStruct((S, D), jnp.float32),
        mesh=mesh,
        scratch_types=[
            pltpu.VMEM((_TPW,), jnp.int32),
            pltpu.VMEM((_TPW,), jnp.int32),
            pltpu.VMEM((_TPW, D), jnp.float32),
            pltpu.VMEM((_TPW, D), jnp.float32),
            pltpu.SemaphoreType.DMA,
            pltpu.SemaphoreType.DMA,
        ],
    )
    def _sc_combine(y_hbm, pos0_hbm, pos1_hbm, out_hbm,
                    idx0_v, idx1_v, rows0_v, rows1_v, s0, s1):
        wid = jax.lax.axis_index("s") * _SC_NC + jax.lax.axis_index("c")
        base = wid * _TPW
        pltpu.sync_copy(pos0_hbm.at[wid], idx0_v)
        pltpu.sync_copy(pos1_hbm.at[wid], idx1_v)
        c0 = pltpu.async_copy(y_hbm.at[idx0_v], rows0_v, s0)
        c1 = pltpu.async_copy(y_hbm.at[idx1_v], rows1_v, s1)
        c0.wait()
        c1.wait()

        def body_i(i, carry):
            for j in range(D // 16):
                sl = pl.ds(j * 16, 16)
                rows0_v[i, sl] = rows0_v[i, sl] + rows1_v[i, sl]
            return carry

        jax.lax.fori_loop(0, _TPW, body_i, 0)
        pltpu.sync_copy(rows0_v, out_hbm.at[pl.ds(base, _TPW)])

    return _sc_dispatch, _sc_combine


def kernel(x, kv_cache, Wg, bg, Wa, ba, W1, b1, W2, b2):
    B = x.shape[0]
    x2 = x.reshape(S, D)
    (pos0, pos1, w0, w1, te, act, first, nxt, do, bufp,
     nxt2, do2) = _routing(x2, Wg, bg)
    pos0w = pos0.reshape(_NW, _TPW)
    pos1w = pos1.reshape(_NW, _TPW)
    sc_dispatch, sc_combine = _sc_kernels()
    xs, rw = sc_dispatch(x2, pos0w, pos1w, w0, w1)
    y = _ffn(xs, rw, te.reshape(NT), act.reshape(NT), first.reshape(NT),
             nxt.reshape(NT), do.reshape(NT), bufp.reshape(NT),
             nxt2.reshape(NT), do2.reshape(NT), Wa, ba, W1, b1, W2, b2)
    out = sc_combine(y, pos0w, pos1w)
    return out.astype(jnp.float16).reshape(B, S, D)


# speedup vs baseline: 1.0238x; 1.0238x over previous
import functools

import jax
import jax.numpy as jnp
from jax.experimental import pallas as pl
from jax.experimental.pallas import tpu as pltpu
from jax.experimental.pallas import tpu_sc as plsc

E = 8
K = 2
D = 768
H = 1536
S = 2048
T = 256
NT = 24
P = NT * T
CHUNK = 256


def _routing_body(x_ref, wg_ref, bg_ref, pos0_ref, pos1_ref, w0_ref, w1_ref,
                  te_ref, act_ref, first_ref, nxt_ref, do_ref, bufp_ref,
                  nxt2_ref, do2_ref):
    x = x_ref[...]
    logits = jnp.dot(x, wg_ref[...], preferred_element_type=jnp.float32)
    logits = logits + bg_ref[...]
    eidx = jax.lax.broadcasted_iota(jnp.int32, (S, E), 1)
    m0 = jnp.max(logits, axis=1, keepdims=True)
    a0 = jnp.argmax(logits, axis=1).reshape(-1, 1)
    masked = jnp.where(eidx == a0, -jnp.inf, logits)
    m1 = jnp.max(masked, axis=1, keepdims=True)
    a1 = jnp.argmax(masked, axis=1).reshape(-1, 1)
    w0 = 1.0 / (1.0 + jnp.exp(m1 - m0))
    w1 = 1.0 - w0
    w0_ref[...] = jnp.broadcast_to(w0, (S, 128))
    w1_ref[...] = jnp.broadcast_to(w1, (S, 128))

    oh0 = (eidx == a0).astype(jnp.float32)
    oh1 = (eidx == a1).astype(jnp.float32)
    ohsum = oh0 + oh1

    r_i = jax.lax.broadcasted_iota(jnp.int32, (CHUNK, CHUNK), 0)
    c_i = jax.lax.broadcasted_iota(jnp.int32, (CHUNK, CHUNK), 1)
    Lt = (c_i < r_i).astype(jnp.float32)
    carry = jnp.zeros((1, E), jnp.float32)
    excs = []
    for i in range(S // CHUNK):
        blk = ohsum[i * CHUNK:(i + 1) * CHUNK, :]
        excs.append(jnp.dot(Lt, blk, preferred_element_type=jnp.float32) + carry)
        carry = carry + jnp.sum(blk, axis=0, keepdims=True)
    exc = jnp.concatenate(excs, axis=0)
    counts = carry

    ci = counts.astype(jnp.int32)
    pc = ((ci + (T - 1)) // T) * T
    e_r = jax.lax.broadcasted_iota(jnp.int32, (E, E), 0)
    e_c = jax.lax.broadcasted_iota(jnp.int32, (E, E), 1)
    base = jnp.sum(jnp.where(e_c < e_r, jnp.broadcast_to(pc, (E, E)), 0),
                   axis=1).reshape(1, E)
    cc = base + pc

    basef = base.astype(jnp.float32)
    pos0 = jnp.sum(oh0 * (basef + exc), axis=1, keepdims=True)
    pos1 = jnp.sum(oh1 * (basef + exc), axis=1, keepdims=True)
    pos0_ref[...] = pos0.astype(jnp.int32)
    pos1_ref[...] = pos1.astype(jnp.int32)

    t_i = jax.lax.broadcasted_iota(jnp.int32, (NT, E), 0) * T
    te = jnp.sum((t_i >= jnp.broadcast_to(cc, (NT, E))).astype(jnp.int32),
                 axis=1, keepdims=True)
    act = (te < E).astype(jnp.int32)
    act_ref[...] = act
    te_ref[...] = jnp.minimum(te, E - 1)

    te_prev = jnp.concatenate([jnp.full((1, 1), -1, jnp.int32), te[:-1]], axis=0)
    first = ((te != te_prev) & (act == 1)).astype(jnp.int32)
    tt_r = jax.lax.broadcasted_iota(jnp.int32, (NT, NT), 0)
    tt_c = jax.lax.broadcasted_iota(jnp.int32, (NT, NT), 1)
    g = jnp.sum(jnp.where(tt_c <= tt_r,
                          jnp.broadcast_to(first.reshape(1, NT), (NT, NT)), 0),
                axis=1, keepdims=True) - 1
    bufp_ref[...] = jnp.maximum(g, 0) % 2
    e_ids = jax.lax.broadcasted_iota(jnp.int32, (NT, E), 1)
    has = jnp.broadcast_to(ci, (NT, E)) > 0
    cand = jnp.where((e_ids > jnp.minimum(te, E - 1)) & has, e_ids, 99)
    nxt = jnp.min(cand, axis=1, keepdims=True)
    cand2 = jnp.where((e_ids > nxt) & has, e_ids, 99)
    nxt2 = jnp.min(cand2, axis=1, keepdims=True)
    do = ((nxt < E) & (first == 1)).astype(jnp.int32)
    do2 = ((nxt2 < E) & (first == 1)).astype(jnp.int32)
    first_ref[...] = first
    do_ref[...] = do
    nxt_ref[...] = jnp.where(nxt < E, nxt, 0)
    do2_ref[...] = do2
    nxt2_ref[...] = jnp.where(nxt2 < E, nxt2, 0)


@functools.partial(jax.jit)
def _routing(x2, Wg, bg):
    return pl.pallas_call(
        _routing_body,
        out_shape=[
            jax.ShapeDtypeStruct((S, 1), jnp.int32),
            jax.ShapeDtypeStruct((S, 1), jnp.int32),
            jax.ShapeDtypeStruct((S, 128), jnp.float32),
            jax.ShapeDtypeStruct((S, 128), jnp.float32),
            jax.ShapeDtypeStruct((NT, 1), jnp.int32),
            jax.ShapeDtypeStruct((NT, 1), jnp.int32),
            jax.ShapeDtypeStruct((NT, 1), jnp.int32),
            jax.ShapeDtypeStruct((NT, 1), jnp.int32),
            jax.ShapeDtypeStruct((NT, 1), jnp.int32),
            jax.ShapeDtypeStruct((NT, 1), jnp.int32),
            jax.ShapeDtypeStruct((NT, 1), jnp.int32),
            jax.ShapeDtypeStruct((NT, 1), jnp.int32),
        ],
    )(x2, Wg, bg)


def _ffn_body(te_ref, act_ref, first_ref, nxt_ref, do_ref, bufp_ref,
              nxt2_ref, do2_ref,
              xs_ref, rw_ref, wa_any, w1_any, w2_any, ba_ref, b1_ref, b2_ref,
              y_ref, wab, w1b, w2b, sa, s1, s2):
    t = pl.program_id(0)
    p = bufp_ref[t]

    def _w_copy(e, slot):
        return (
            pltpu.make_async_copy(wa_any.at[e], wab.at[slot], sa.at[slot]),
            pltpu.make_async_copy(w1_any.at[e], w1b.at[slot], s1.at[slot]),
            pltpu.make_async_copy(w2_any.at[e], w2b.at[slot], s2.at[slot]),
        )

    @pl.when(t == 0)
    def _prime():
        for c in _w_copy(te_ref[0], 0):
            c.start()

    @pl.when(first_ref[t] == 1)
    def _stream():
        for c in _w_copy(te_ref[t], p):
            c.wait()

        @pl.when(do_ref[t] == 1)
        def _prefetch():
            for c in _w_copy(nxt_ref[t], 1 - p):
                c.start()

    @pl.when(act_ref[t] == 1)
    def _go():
        x = xs_ref[...]
        e = te_ref[t]
        a = jnp.dot(x, wab[p], preferred_element_type=jnp.float32) + ba_ref[e]
        a = a * jax.nn.sigmoid(a)
        f1 = jnp.dot(x, w1b[p], preferred_element_type=jnp.float32) + b1_ref[e]
        h = a * f1
        o = jnp.dot(h, w2b[p], preferred_element_type=jnp.float32) + b2_ref[e]
        y_ref[...] = o * rw_ref[...][:, 0:1]


@functools.partial(jax.jit)
def _ffn(xs, rw, te, act, first, nxt, do, bufp, nxt2, do2,
         Wa, ba, W1, b1, W2, b2):
    grid_spec = pltpu.PrefetchScalarGridSpec(
        num_scalar_prefetch=8,
        grid=(NT,),
        in_specs=[
            pl.BlockSpec((T, D), lambda t, *_: (t, 0)),
            pl.BlockSpec((T, 128), lambda t, *_: (t, 0)),
            pl.BlockSpec(memory_space=pl.ANY),
            pl.BlockSpec(memory_space=pl.ANY),
            pl.BlockSpec(memory_space=pl.ANY),
            pl.BlockSpec((E, 1, H), lambda t, *_: (0, 0, 0)),
            pl.BlockSpec((E, 1, H), lambda t, *_: (0, 0, 0)),
            pl.BlockSpec((E, 1, D), lambda t, *_: (0, 0, 0)),
        ],
        out_specs=pl.BlockSpec((T, D), lambda t, *_: (t, 0)),
        scratch_shapes=[
            pltpu.VMEM((2, D, H), jnp.float32),
            pltpu.VMEM((2, D, H), jnp.float32),
            pltpu.VMEM((2, H, D), jnp.float32),
            pltpu.SemaphoreType.DMA((2,)),
            pltpu.SemaphoreType.DMA((2,)),
            pltpu.SemaphoreType.DMA((2,)),
        ],
    )
    return pl.pallas_call(
        _ffn_body,
        grid_spec=grid_spec,
        out_shape=jax.ShapeDtypeStruct((P, D), jnp.float32),
        compiler_params=pltpu.CompilerParams(
            dimension_semantics=("arbitrary",),
        ),
    )(te, act, first, nxt, do, bufp, nxt2, do2, xs, rw, Wa, W1, W2,
      ba.reshape(E, 1, H), b1.reshape(E, 1, H), b2.reshape(E, 1, D))


_SC_NC = 2
_SC_NS = 16
_NW = _SC_NC * _SC_NS
_TPW = S // _NW


@functools.cache
def _sc_kernels():
    mesh = plsc.VectorSubcoreMesh(core_axis_name="c", subcore_axis_name="s",
                                  num_cores=_SC_NC, num_subcores=_SC_NS)

    @functools.partial(
        pl.kernel,
        out_type=[
            jax.ShapeDtypeStruct((P, D), jnp.float32),
            jax.ShapeDtypeStruct((P, 128), jnp.float32),
        ],
        mesh=mesh,
        scratch_types=[
            pltpu.VMEM((_TPW,), jnp.int32),
            pltpu.VMEM((_TPW,), jnp.int32),
            pltpu.VMEM((_TPW, D), jnp.float32),
            pltpu.VMEM((_TPW, 128), jnp.float32),
            pltpu.VMEM((_TPW, 128), jnp.float32),
            pltpu.SemaphoreType.DMA,
            pltpu.SemaphoreType.DMA,
            pltpu.SemaphoreType.DMA,
            pltpu.SemaphoreType.DMA,
        ],
    )
    def _sc_dispatch(x_hbm, pos0_hbm, pos1_hbm, w0_hbm, w1_hbm, xs_hbm, rw_hbm,
                     idx0_v, idx1_v, rows_v, w0_v, w1_v, s0, s1, s2, s3):
        wid = jax.lax.axis_index("s") * _SC_NC + jax.lax.axis_index("c")
        base = wid * _TPW
        pltpu.sync_copy(pos0_hbm.at[wid], idx0_v)
        pltpu.sync_copy(pos1_hbm.at[wid], idx1_v)
        pltpu.sync_copy(x_hbm.at[pl.ds(base, _TPW)], rows_v)
        pltpu.sync_copy(w0_hbm.at[pl.ds(base, _TPW)], w0_v)
        pltpu.sync_copy(w1_hbm.at[pl.ds(base, _TPW)], w1_v)
        c0 = pltpu.async_copy(rows_v, xs_hbm.at[idx0_v], s0)
        c1 = pltpu.async_copy(rows_v, xs_hbm.at[idx1_v], s1)
        c2 = pltpu.async_copy(w0_v, rw_hbm.at[idx0_v], s2)
        c3 = pltpu.async_copy(w1_v, rw_hbm.at[idx1_v], s3)
        c0.wait()
        c1.wait()
        c2.wait()
        c3.wait()


    @functools.partial(
        pl.kernel,
        out_type=jax.ShapeDtypeStruct((S, D), jnp.float32),
        mesh=mesh,
        scratch_types=[
            pltpu.VMEM((_TPW,), jnp.int32),
            pltpu.VMEM((_TPW,), jnp.int32),
            pltpu.VMEM((_TPW, D), jnp.float32),
            pltpu.VMEM((_TPW, D), jnp.float32),
            pltpu.SemaphoreType.DMA,
            pltpu.SemaphoreType.DMA,
        ],
    )
    def _sc_combine(y_hbm, pos0_hbm, pos1_hbm, out_hbm,
                    idx0_v, idx1_v, rows0_v, rows1_v, s0, s1):
        wid = jax.lax.axis_index("s") * _SC_NC + jax.lax.axis_index("c")
        base = wid * _TPW
        pltpu.sync_copy(pos0_hbm.at[wid], idx0_v)
        pltpu.sync_copy(pos1_hbm.at[wid], idx1_v)
        c0 = pltpu.async_copy(y_hbm.at[idx0_v], rows0_v, s0)
        c1 = pltpu.async_copy(y_hbm.at[idx1_v], rows1_v, s1)
        c0.wait()
        c1.wait()

        def body_i(i, carry):
            for j in range(D // 16):
                sl = pl.ds(j * 16, 16)
                rows0_v[i, sl] = rows0_v[i, sl] + rows1_v[i, sl]
            return carry

        jax.lax.fori_loop(0, _TPW, body_i, 0)
        pltpu.sync_copy(rows0_v, out_hbm.at[pl.ds(base, _TPW)])

    return _sc_dispatch, _sc_combine


def kernel(x, kv_cache, Wg, bg, Wa, ba, W1, b1, W2, b2):
    B = x.shape[0]
    x2 = x.reshape(S, D)
    (pos0, pos1, w0, w1, te, act, first, nxt, do, bufp,
     nxt2, do2) = _routing(x2, Wg, bg)
    pos0w = pos0.reshape(_NW, _TPW)
    pos1w = pos1.reshape(_NW, _TPW)
    sc_dispatch, sc_combine = _sc_kernels()
    xs, rw = sc_dispatch(x2, pos0w, pos1w, w0, w1)
    y = _ffn(xs, rw, te.reshape(NT), act.reshape(NT), first.reshape(NT),
             nxt.reshape(NT), do.reshape(NT), bufp.reshape(NT),
             nxt2.reshape(NT), do2.reshape(NT), Wa, ba, W1, b1, W2, b2)
    out = sc_combine(y, pos0w, pos1w)
    return out.astype(jnp.float16).reshape(B, S, D)


# speedup vs baseline: 1.0550x; 1.0304x over previous
import functools

import jax
import jax.numpy as jnp
from jax.experimental import pallas as pl
from jax.experimental.pallas import tpu as pltpu
from jax.experimental.pallas import tpu_sc as plsc

E = 8
K = 2
D = 768
H = 1536
S = 2048
T = 256
NT = 24
P = NT * T
CHUNK = 256


def _routing_body(x_ref, wg_ref, bg_ref, pos0_ref, pos1_ref, w0_ref, w1_ref,
                  te_ref, act_ref, first_ref, nxt_ref, do_ref, bufp_ref,
                  nxt2_ref, do2_ref):
    x = x_ref[...]
    logits = jnp.dot(x, wg_ref[...], preferred_element_type=jnp.float32)
    logits = logits + bg_ref[...]
    eidx = jax.lax.broadcasted_iota(jnp.int32, (S, E), 1)
    m0 = jnp.max(logits, axis=1, keepdims=True)
    a0 = jnp.argmax(logits, axis=1).reshape(-1, 1)
    masked = jnp.where(eidx == a0, -jnp.inf, logits)
    m1 = jnp.max(masked, axis=1, keepdims=True)
    a1 = jnp.argmax(masked, axis=1).reshape(-1, 1)
    w0 = 1.0 / (1.0 + jnp.exp(m1 - m0))
    w1 = 1.0 - w0
    w0_ref[...] = jnp.broadcast_to(w0, (S, 128))
    w1_ref[...] = jnp.broadcast_to(w1, (S, 128))

    oh0 = (eidx == a0).astype(jnp.float32)
    oh1 = (eidx == a1).astype(jnp.float32)
    ohsum = oh0 + oh1

    r_i = jax.lax.broadcasted_iota(jnp.int32, (CHUNK, CHUNK), 0)
    c_i = jax.lax.broadcasted_iota(jnp.int32, (CHUNK, CHUNK), 1)
    Lt = (c_i < r_i).astype(jnp.float32)
    carry = jnp.zeros((1, E), jnp.float32)
    excs = []
    for i in range(S // CHUNK):
        blk = ohsum[i * CHUNK:(i + 1) * CHUNK, :]
        excs.append(jnp.dot(Lt, blk, preferred_element_type=jnp.float32) + carry)
        carry = carry + jnp.sum(blk, axis=0, keepdims=True)
    exc = jnp.concatenate(excs, axis=0)
    counts = carry

    ci = counts.astype(jnp.int32)
    pc = ((ci + (T - 1)) // T) * T
    e_r = jax.lax.broadcasted_iota(jnp.int32, (E, E), 0)
    e_c = jax.lax.broadcasted_iota(jnp.int32, (E, E), 1)
    base = jnp.sum(jnp.where(e_c < e_r, jnp.broadcast_to(pc, (E, E)), 0),
                   axis=1).reshape(1, E)
    cc = base + pc

    basef = base.astype(jnp.float32)
    pos0 = jnp.sum(oh0 * (basef + exc), axis=1, keepdims=True)
    pos1 = jnp.sum(oh1 * (basef + exc), axis=1, keepdims=True)
    pos0_ref[...] = pos0.astype(jnp.int32)
    pos1_ref[...] = pos1.astype(jnp.int32)

    t_i = jax.lax.broadcasted_iota(jnp.int32, (NT, E), 0) * T
    te = jnp.sum((t_i >= jnp.broadcast_to(cc, (NT, E))).astype(jnp.int32),
                 axis=1, keepdims=True)
    act = (te < E).astype(jnp.int32)
    act_ref[...] = act
    te_ref[...] = jnp.minimum(te, E - 1)

    te_prev = jnp.concatenate([jnp.full((1, 1), -1, jnp.int32), te[:-1]], axis=0)
    first = ((te != te_prev) & (act == 1)).astype(jnp.int32)
    tt_r = jax.lax.broadcasted_iota(jnp.int32, (NT, NT), 0)
    tt_c = jax.lax.broadcasted_iota(jnp.int32, (NT, NT), 1)
    g = jnp.sum(jnp.where(tt_c <= tt_r,
                          jnp.broadcast_to(first.reshape(1, NT), (NT, NT)), 0),
                axis=1, keepdims=True) - 1
    bufp_ref[...] = jnp.maximum(g, 0) % 2
    e_ids = jax.lax.broadcasted_iota(jnp.int32, (NT, E), 1)
    has = jnp.broadcast_to(ci, (NT, E)) > 0
    cand = jnp.where((e_ids > jnp.minimum(te, E - 1)) & has, e_ids, 99)
    nxt = jnp.min(cand, axis=1, keepdims=True)
    cand2 = jnp.where((e_ids > nxt) & has, e_ids, 99)
    nxt2 = jnp.min(cand2, axis=1, keepdims=True)
    do = ((nxt < E) & (first == 1)).astype(jnp.int32)
    do2 = ((nxt2 < E) & (first == 1)).astype(jnp.int32)
    first_ref[...] = first
    do_ref[...] = do
    nxt_ref[...] = jnp.where(nxt < E, nxt, 0)
    do2_ref[...] = do2
    nxt2_ref[...] = jnp.where(nxt2 < E, nxt2, 0)


@functools.partial(jax.jit)
def _routing(x2, Wg, bg):
    return pl.pallas_call(
        _routing_body,
        out_shape=[
            jax.ShapeDtypeStruct((S, 1), jnp.int32),
            jax.ShapeDtypeStruct((S, 1), jnp.int32),
            jax.ShapeDtypeStruct((S, 128), jnp.float32),
            jax.ShapeDtypeStruct((S, 128), jnp.float32),
            jax.ShapeDtypeStruct((NT, 1), jnp.int32),
            jax.ShapeDtypeStruct((NT, 1), jnp.int32),
            jax.ShapeDtypeStruct((NT, 1), jnp.int32),
            jax.ShapeDtypeStruct((NT, 1), jnp.int32),
            jax.ShapeDtypeStruct((NT, 1), jnp.int32),
            jax.ShapeDtypeStruct((NT, 1), jnp.int32),
            jax.ShapeDtypeStruct((NT, 1), jnp.int32),
            jax.ShapeDtypeStruct((NT, 1), jnp.int32),
        ],
    )(x2, Wg, bg)


def _ffn_body(te_ref, act_ref, first_ref, nxt_ref, do_ref, bufp_ref,
              nxt2_ref, do2_ref,
              xs_ref, rw_ref, wa_any, w1_any, w2_any, ba_ref, b1_ref, b2_ref,
              y_ref, wab, w1b, w2b, sa, s1, s2):
    t = pl.program_id(0)
    p = bufp_ref[t]

    def _w_copy(e, slot):
        return (
            pltpu.make_async_copy(wa_any.at[e], wab.at[slot], sa.at[slot]),
            pltpu.make_async_copy(w1_any.at[e], w1b.at[slot], s1.at[slot]),
            pltpu.make_async_copy(w2_any.at[e], w2b.at[slot], s2.at[slot]),
        )

    @pl.when(t == 0)
    def _prime():
        for c in _w_copy(te_ref[0], 0):
            c.start()

    @pl.when(first_ref[t] == 1)
    def _stream():
        for c in _w_copy(te_ref[t], p):
            c.wait()

        @pl.when(do_ref[t] == 1)
        def _prefetch():
            for c in _w_copy(nxt_ref[t], 1 - p):
                c.start()

    @pl.when(act_ref[t] == 1)
    def _go():
        x = xs_ref[...]
        e = te_ref[t]
        a = jnp.dot(x, wab[p], preferred_element_type=jnp.float32) + ba_ref[e]
        a = a * jax.nn.sigmoid(a)
        f1 = jnp.dot(x, w1b[p], preferred_element_type=jnp.float32) + b1_ref[e]
        h = a * f1
        o = jnp.dot(h, w2b[p], preferred_element_type=jnp.float32) + b2_ref[e]
        y_ref[...] = o * rw_ref[...][:, 0:1]


@functools.partial(jax.jit)
def _ffn(xs, rw, te, act, first, nxt, do, bufp, nxt2, do2,
         Wa, ba, W1, b1, W2, b2):
    grid_spec = pltpu.PrefetchScalarGridSpec(
        num_scalar_prefetch=8,
        grid=(NT,),
        in_specs=[
            pl.BlockSpec((T, D),
                         lambda t, te, act, *_: (jnp.where(act[t] == 1, t, 0), 0)),
            pl.BlockSpec((T, 128),
                         lambda t, te, act, *_: (jnp.where(act[t] == 1, t, 0), 0)),
            pl.BlockSpec(memory_space=pl.ANY),
            pl.BlockSpec(memory_space=pl.ANY),
            pl.BlockSpec(memory_space=pl.ANY),
            pl.BlockSpec((E, 1, H), lambda t, *_: (0, 0, 0)),
            pl.BlockSpec((E, 1, H), lambda t, *_: (0, 0, 0)),
            pl.BlockSpec((E, 1, D), lambda t, *_: (0, 0, 0)),
        ],
        out_specs=pl.BlockSpec(
            (T, D), lambda t, te, act, *_: (jnp.where(act[t] == 1, t, NT - 1), 0)),
        scratch_shapes=[
            pltpu.VMEM((2, D, H), jnp.float32),
            pltpu.VMEM((2, D, H), jnp.float32),
            pltpu.VMEM((2, H, D), jnp.float32),
            pltpu.SemaphoreType.DMA((2,)),
            pltpu.SemaphoreType.DMA((2,)),
            pltpu.SemaphoreType.DMA((2,)),
        ],
    )
    return pl.pallas_call(
        _ffn_body,
        grid_spec=grid_spec,
        out_shape=jax.ShapeDtypeStruct((P, D), jnp.float32),
        compiler_params=pltpu.CompilerParams(
            dimension_semantics=("arbitrary",),
        ),
    )(te, act, first, nxt, do, bufp, nxt2, do2, xs, rw, Wa, W1, W2,
      ba.reshape(E, 1, H), b1.reshape(E, 1, H), b2.reshape(E, 1, D))


_SC_NC = 2
_SC_NS = 16
_NW = _SC_NC * _SC_NS
_TPW = S // _NW


@functools.cache
def _sc_kernels():
    mesh = plsc.VectorSubcoreMesh(core_axis_name="c", subcore_axis_name="s",
                                  num_cores=_SC_NC, num_subcores=_SC_NS)

    @functools.partial(
        pl.kernel,
        out_type=[
            jax.ShapeDtypeStruct((P, D), jnp.float32),
            jax.ShapeDtypeStruct((P, 128), jnp.float32),
        ],
        mesh=mesh,
        scratch_types=[
            pltpu.VMEM((_TPW,), jnp.int32),
            pltpu.VMEM((_TPW,), jnp.int32),
            pltpu.VMEM((_TPW, D), jnp.float32),
            pltpu.VMEM((_TPW, 128), jnp.float32),
            pltpu.VMEM((_TPW, 128), jnp.float32),
            pltpu.SemaphoreType.DMA,
            pltpu.SemaphoreType.DMA,
            pltpu.SemaphoreType.DMA,
            pltpu.SemaphoreType.DMA,
        ],
    )
    def _sc_dispatch(x_hbm, pos0_hbm, pos1_hbm, w0_hbm, w1_hbm, xs_hbm, rw_hbm,
                     idx0_v, idx1_v, rows_v, w0_v, w1_v, s0, s1, s2, s3):
        wid = jax.lax.axis_index("s") * _SC_NC + jax.lax.axis_index("c")
        base = wid * _TPW
        pltpu.sync_copy(pos0_hbm.at[wid], idx0_v)
        pltpu.sync_copy(pos1_hbm.at[wid], idx1_v)
        pltpu.sync_copy(x_hbm.at[pl.ds(base, _TPW)], rows_v)
        pltpu.sync_copy(w0_hbm.at[pl.ds(base, _TPW)], w0_v)
        pltpu.sync_copy(w1_hbm.at[pl.ds(base, _TPW)], w1_v)
        c0 = pltpu.async_copy(rows_v, xs_hbm.at[idx0_v], s0)
        c1 = pltpu.async_copy(rows_v, xs_hbm.at[idx1_v], s1)
        c2 = pltpu.async_copy(w0_v, rw_hbm.at[idx0_v], s2)
        c3 = pltpu.async_copy(w1_v, rw_hbm.at[idx1_v], s3)
        c0.wait()
        c1.wait()
        c2.wait()
        c3.wait()


    @functools.partial(
        pl.kernel,
        out_type=jax.ShapeDtypeStruct((S, D), jnp.float32),
        mesh=mesh,
        scratch_types=[
            pltpu.VMEM((_TPW,), jnp.int32),
            pltpu.VMEM((_TPW,), jnp.int32),
            pltpu.VMEM((_TPW, D), jnp.float32),
            pltpu.VMEM((_TPW, D), jnp.float32),
            pltpu.SemaphoreType.DMA,
            pltpu.SemaphoreType.DMA,
        ],
    )
    def _sc_combine(y_hbm, pos0_hbm, pos1_hbm, out_hbm,
                    idx0_v, idx1_v, rows0_v, rows1_v, s0, s1):
        wid = jax.lax.axis_index("s") * _SC_NC + jax.lax.axis_index("c")
        base = wid * _TPW
        pltpu.sync_copy(pos0_hbm.at[wid], idx0_v)
        pltpu.sync_copy(pos1_hbm.at[wid], idx1_v)
        c0 = pltpu.async_copy(y_hbm.at[idx0_v], rows0_v, s0)
        c1 = pltpu.async_copy(y_hbm.at[idx1_v], rows1_v, s1)
        c0.wait()
        c1.wait()

        def body_i(i, carry):
            for j in range(D // 16):
                sl = pl.ds(j * 16, 16)
                rows0_v[i, sl] = rows0_v[i, sl] + rows1_v[i, sl]
            return carry

        jax.lax.fori_loop(0, _TPW, body_i, 0)
        pltpu.sync_copy(rows0_v, out_hbm.at[pl.ds(base, _TPW)])

    return _sc_dispatch, _sc_combine


def kernel(x, kv_cache, Wg, bg, Wa, ba, W1, b1, W2, b2):
    B = x.shape[0]
    x2 = x.reshape(S, D)
    (pos0, pos1, w0, w1, te, act, first, nxt, do, bufp,
     nxt2, do2) = _routing(x2, Wg, bg)
    pos0w = pos0.reshape(_NW, _TPW)
    pos1w = pos1.reshape(_NW, _TPW)
    sc_dispatch, sc_combine = _sc_kernels()
    xs, rw = sc_dispatch(x2, pos0w, pos1w, w0, w1)
    y = _ffn(xs, rw, te.reshape(NT), act.reshape(NT), first.reshape(NT),
             nxt.reshape(NT), do.reshape(NT), bufp.reshape(NT),
             nxt2.reshape(NT), do2.reshape(NT), Wa, ba, W1, b1, W2, b2)
    out = sc_combine(y, pos0w, pos1w)
    return out.astype(jnp.float16).reshape(B, S, D)


# speedup vs baseline: 1.0601x; 1.0048x over previous
import functools

import jax
import jax.numpy as jnp
from jax.experimental import pallas as pl
from jax.experimental.pallas import tpu as pltpu
from jax.experimental.pallas import tpu_sc as plsc

E = 8
K = 2
D = 768
H = 1536
S = 2048
T = 256
NT = 24
P = NT * T
CHUNK = 256


def _routing_body(x_ref, wg_ref, bg_ref, pos0_ref, pos1_ref, w0_ref, w1_ref,
                  te_ref, act_ref, first_ref, nxt_ref, do_ref, bufp_ref,
                  nxt2_ref, do2_ref):
    x = x_ref[...]
    logits = jnp.dot(x, wg_ref[...], preferred_element_type=jnp.float32)
    logits = logits + bg_ref[...]
    eidx = jax.lax.broadcasted_iota(jnp.int32, (S, E), 1)
    m0 = jnp.max(logits, axis=1, keepdims=True)
    a0 = jnp.argmax(logits, axis=1).reshape(-1, 1)
    masked = jnp.where(eidx == a0, -jnp.inf, logits)
    m1 = jnp.max(masked, axis=1, keepdims=True)
    a1 = jnp.argmax(masked, axis=1).reshape(-1, 1)
    w0 = 1.0 / (1.0 + jnp.exp(m1 - m0))
    w1 = 1.0 - w0
    w0_ref[...] = jnp.broadcast_to(w0, (S, 128))
    w1_ref[...] = jnp.broadcast_to(w1, (S, 128))

    oh0 = (eidx == a0).astype(jnp.float32)
    oh1 = (eidx == a1).astype(jnp.float32)
    ohsum = oh0 + oh1

    r_i = jax.lax.broadcasted_iota(jnp.int32, (CHUNK, CHUNK), 0)
    c_i = jax.lax.broadcasted_iota(jnp.int32, (CHUNK, CHUNK), 1)
    Lt = (c_i < r_i).astype(jnp.float32)
    carry = jnp.zeros((1, E), jnp.float32)
    excs = []
    for i in range(S // CHUNK):
        blk = ohsum[i * CHUNK:(i + 1) * CHUNK, :]
        excs.append(jnp.dot(Lt, blk, preferred_element_type=jnp.float32) + carry)
        carry = carry + jnp.sum(blk, axis=0, keepdims=True)
    exc = jnp.concatenate(excs, axis=0)
    counts = carry

    ci = counts.astype(jnp.int32)
    pc = ((ci + (T - 1)) // T) * T
    e_r = jax.lax.broadcasted_iota(jnp.int32, (E, E), 0)
    e_c = jax.lax.broadcasted_iota(jnp.int32, (E, E), 1)
    base = jnp.sum(jnp.where(e_c < e_r, jnp.broadcast_to(pc, (E, E)), 0),
                   axis=1).reshape(1, E)
    cc = base + pc

    basef = base.astype(jnp.float32)
    pos0 = jnp.sum(oh0 * (basef + exc), axis=1, keepdims=True)
    pos1 = jnp.sum(oh1 * (basef + exc), axis=1, keepdims=True)
    pos0_ref[...] = pos0.astype(jnp.int32)
    pos1_ref[...] = pos1.astype(jnp.int32)

    t_i = jax.lax.broadcasted_iota(jnp.int32, (NT, E), 0) * T
    te = jnp.sum((t_i >= jnp.broadcast_to(cc, (NT, E))).astype(jnp.int32),
                 axis=1, keepdims=True)
    act = (te < E).astype(jnp.int32)
    act_ref[...] = act
    te_ref[...] = jnp.minimum(te, E - 1)

    te_prev = jnp.concatenate([jnp.full((1, 1), -1, jnp.int32), te[:-1]], axis=0)
    first = ((te != te_prev) & (act == 1)).astype(jnp.int32)
    tt_r = jax.lax.broadcasted_iota(jnp.int32, (NT, NT), 0)
    tt_c = jax.lax.broadcasted_iota(jnp.int32, (NT, NT), 1)
    g = jnp.sum(jnp.where(tt_c <= tt_r,
                          jnp.broadcast_to(first.reshape(1, NT), (NT, NT)), 0),
                axis=1, keepdims=True) - 1
    bufp_ref[...] = jnp.maximum(g, 0) % 2
    e_ids = jax.lax.broadcasted_iota(jnp.int32, (NT, E), 1)
    has = jnp.broadcast_to(ci, (NT, E)) > 0
    cand = jnp.where((e_ids > jnp.minimum(te, E - 1)) & has, e_ids, 99)
    nxt = jnp.min(cand, axis=1, keepdims=True)
    cand2 = jnp.where((e_ids > nxt) & has, e_ids, 99)
    nxt2 = jnp.min(cand2, axis=1, keepdims=True)
    do = ((nxt < E) & (first == 1)).astype(jnp.int32)
    do2 = ((nxt2 < E) & (first == 1)).astype(jnp.int32)
    first_ref[...] = first
    do_ref[...] = do
    nxt_ref[...] = jnp.where(nxt < E, nxt, 0)
    do2_ref[...] = do2
    nxt2_ref[...] = jnp.where(nxt2 < E, nxt2, 0)


@functools.partial(jax.jit)
def _routing(x2, Wg, bg):
    return pl.pallas_call(
        _routing_body,
        out_shape=[
            jax.ShapeDtypeStruct((S, 1), jnp.int32),
            jax.ShapeDtypeStruct((S, 1), jnp.int32),
            jax.ShapeDtypeStruct((S, 128), jnp.float32),
            jax.ShapeDtypeStruct((S, 128), jnp.float32),
            jax.ShapeDtypeStruct((NT, 1), jnp.int32),
            jax.ShapeDtypeStruct((NT, 1), jnp.int32),
            jax.ShapeDtypeStruct((NT, 1), jnp.int32),
            jax.ShapeDtypeStruct((NT, 1), jnp.int32),
            jax.ShapeDtypeStruct((NT, 1), jnp.int32),
            jax.ShapeDtypeStruct((NT, 1), jnp.int32),
            jax.ShapeDtypeStruct((NT, 1), jnp.int32),
            jax.ShapeDtypeStruct((NT, 1), jnp.int32),
        ],
    )(x2, Wg, bg)


def _ffn_body(te_ref, act_ref, first_ref, nxt_ref, do_ref, bufp_ref,
              nxt2_ref, do2_ref,
              xs_ref, rw_ref, wa_any, w1_any, w2_any, ba_ref, b1_ref, b2_ref,
              y_ref, wab, w1b, w2b, sa, s1, s2):
    t = pl.program_id(0)
    p = bufp_ref[t]

    def _w_copy(e, slot):
        return (
            pltpu.make_async_copy(wa_any.at[e], wab.at[slot], sa.at[slot]),
            pltpu.make_async_copy(w1_any.at[e], w1b.at[slot], s1.at[slot]),
            pltpu.make_async_copy(w2_any.at[e], w2b.at[slot], s2.at[slot]),
        )

    @pl.when(t == 0)
    def _prime():
        for c in _w_copy(te_ref[0], 0):
            c.start()

    @pl.when(first_ref[t] == 1)
    def _stream():
        for c in _w_copy(te_ref[t], p):
            c.wait()

        @pl.when(do_ref[t] == 1)
        def _prefetch():
            for c in _w_copy(nxt_ref[t], 1 - p):
                c.start()

    @pl.when(act_ref[t] == 1)
    def _go():
        x = xs_ref[...]
        e = te_ref[t]
        a = jnp.dot(x, wab[p], preferred_element_type=jnp.float32) + ba_ref[e]
        a = a * jax.nn.sigmoid(a)
        f1 = jnp.dot(x, w1b[p], preferred_element_type=jnp.float32) + b1_ref[e]
        h = a * f1
        o = jnp.dot(h, w2b[p], preferred_element_type=jnp.float32) + b2_ref[e]
        y_ref[...] = o * rw_ref[...][:, 0:1]


@functools.partial(jax.jit)
def _ffn(xs, rw, te, act, first, nxt, do, bufp, nxt2, do2,
         Wa, ba, W1, b1, W2, b2):
    grid_spec = pltpu.PrefetchScalarGridSpec(
        num_scalar_prefetch=8,
        grid=(NT,),
        in_specs=[
            pl.BlockSpec((T, D),
                         lambda t, te, act, *_: (jnp.where(act[t] == 1, t, 0), 0)),
            pl.BlockSpec((T, 128),
                         lambda t, te, act, *_: (jnp.where(act[t] == 1, t, 0), 0)),
            pl.BlockSpec(memory_space=pl.ANY),
            pl.BlockSpec(memory_space=pl.ANY),
            pl.BlockSpec(memory_space=pl.ANY),
            pl.BlockSpec((E, 1, H), lambda t, *_: (0, 0, 0)),
            pl.BlockSpec((E, 1, H), lambda t, *_: (0, 0, 0)),
            pl.BlockSpec((E, 1, D), lambda t, *_: (0, 0, 0)),
        ],
        out_specs=pl.BlockSpec(
            (T, D), lambda t, te, act, *_: (jnp.where(act[t] == 1, t, NT - 1), 0)),
        scratch_shapes=[
            pltpu.VMEM((2, D, H), jnp.float32),
            pltpu.VMEM((2, D, H), jnp.float32),
            pltpu.VMEM((2, H, D), jnp.float32),
            pltpu.SemaphoreType.DMA((2,)),
            pltpu.SemaphoreType.DMA((2,)),
            pltpu.SemaphoreType.DMA((2,)),
        ],
    )
    return pl.pallas_call(
        _ffn_body,
        grid_spec=grid_spec,
        out_shape=jax.ShapeDtypeStruct((P, D), jnp.float32),
        compiler_params=pltpu.CompilerParams(
            dimension_semantics=("arbitrary",),
        ),
    )(te, act, first, nxt, do, bufp, nxt2, do2, xs, rw, Wa, W1, W2,
      ba.reshape(E, 1, H), b1.reshape(E, 1, H), b2.reshape(E, 1, D))


_SC_NC = 2
_SC_NS = 16
_NW = _SC_NC * _SC_NS
_TPW = S // _NW


@functools.cache
def _sc_kernels():
    mesh = plsc.VectorSubcoreMesh(core_axis_name="c", subcore_axis_name="s",
                                  num_cores=_SC_NC, num_subcores=_SC_NS)

    @functools.partial(
        pl.kernel,
        out_type=[
            jax.ShapeDtypeStruct((P, D), jnp.float32),
            jax.ShapeDtypeStruct((P, 128), jnp.float32),
        ],
        mesh=mesh,
        scratch_types=[
            pltpu.VMEM((_TPW // 16, 16), jnp.int32),
            pltpu.VMEM((_TPW // 16, 16), jnp.int32),
            pltpu.VMEM((_TPW, D), jnp.float32),
            pltpu.VMEM((_TPW, 128), jnp.float32),
            pltpu.VMEM((_TPW, 128), jnp.float32),
            pltpu.SemaphoreType.DMA((4,)),
            pltpu.SemaphoreType.DMA((4,)),
            pltpu.SemaphoreType.DMA((4,)),
            pltpu.SemaphoreType.DMA((4,)),
            pltpu.SemaphoreType.DMA((4,)),
        ],
    )
    def _sc_dispatch(x_hbm, pos0_hbm, pos1_hbm, w0_hbm, w1_hbm, xs_hbm, rw_hbm,
                     idx0_v, idx1_v, rows_v, w0_v, w1_v, s0, s1, s2, s3, sl):
        wid = jax.lax.axis_index("s") * _SC_NC + jax.lax.axis_index("c")
        base = wid * _TPW
        CH = 16
        NCH = _TPW // CH
        pltpu.sync_copy(pos0_hbm.at[wid], idx0_v)
        pltpu.sync_copy(pos1_hbm.at[wid], idx1_v)
        for c in range(NCH):
            r = pl.ds(c * CH, CH)
            pltpu.async_copy(x_hbm.at[pl.ds(base + c * CH, CH)], rows_v.at[r],
                             sl.at[c])
        pltpu.sync_copy(w0_hbm.at[pl.ds(base, _TPW)], w0_v)
        pltpu.sync_copy(w1_hbm.at[pl.ds(base, _TPW)], w1_v)
        for c in range(NCH):
            r = pl.ds(c * CH, CH)
            pltpu.async_copy(w0_v.at[r], rw_hbm.at[idx0_v.at[c]], s2.at[c])
            pltpu.async_copy(w1_v.at[r], rw_hbm.at[idx1_v.at[c]], s3.at[c])
            pltpu.make_async_copy(x_hbm.at[pl.ds(base + c * CH, CH)],
                                  rows_v.at[r], sl.at[c]).wait()
            pltpu.async_copy(rows_v.at[r], xs_hbm.at[idx0_v.at[c]], s0.at[c])
            pltpu.async_copy(rows_v.at[r], xs_hbm.at[idx1_v.at[c]], s1.at[c])
        for c in range(NCH):
            r = pl.ds(c * CH, CH)
            pltpu.make_async_copy(w0_v.at[r], rw_hbm.at[idx0_v.at[c]],
                                  s2.at[c]).wait()
            pltpu.make_async_copy(w1_v.at[r], rw_hbm.at[idx1_v.at[c]],
                                  s3.at[c]).wait()
            pltpu.make_async_copy(rows_v.at[r], xs_hbm.at[idx0_v.at[c]],
                                  s0.at[c]).wait()
            pltpu.make_async_copy(rows_v.at[r], xs_hbm.at[idx1_v.at[c]],
                                  s1.at[c]).wait()


    @functools.partial(
        pl.kernel,
        out_type=jax.ShapeDtypeStruct((S, D), jnp.float32),
        mesh=mesh,
        scratch_types=[
            pltpu.VMEM((_TPW,), jnp.int32),
            pltpu.VMEM((_TPW,), jnp.int32),
            pltpu.VMEM((_TPW, D), jnp.float32),
            pltpu.VMEM((_TPW, D), jnp.float32),
            pltpu.SemaphoreType.DMA((4,)),
            pltpu.SemaphoreType.DMA((4,)),
            pltpu.SemaphoreType.DMA((4,)),
        ],
    )
    def _sc_combine(y_hbm, pos0_hbm, pos1_hbm, out_hbm,
                    idx0_v, idx1_v, rows0_v, rows1_v, s0, s1, s2):
        wid = jax.lax.axis_index("s") * _SC_NC + jax.lax.axis_index("c")
        base = wid * _TPW
        pltpu.sync_copy(pos0_hbm.at[wid], idx0_v)
        pltpu.sync_copy(pos1_hbm.at[wid], idx1_v)
        CH = 16
        NCH = _TPW // CH
        for c in range(NCH):
            r = pl.ds(c * CH, CH)
            pltpu.async_copy(y_hbm.at[idx0_v.at[r]], rows0_v.at[r], s0.at[c])
            pltpu.async_copy(y_hbm.at[idx1_v.at[r]], rows1_v.at[r], s1.at[c])
        for c in range(NCH):
            r = pl.ds(c * CH, CH)
            pltpu.make_async_copy(y_hbm.at[idx0_v.at[r]], rows0_v.at[r],
                                  s0.at[c]).wait()
            pltpu.make_async_copy(y_hbm.at[idx1_v.at[r]], rows1_v.at[r],
                                  s1.at[c]).wait()

            def body_i(i, carry):
                for j in range(D // 16):
                    sl = pl.ds(j * 16, 16)
                    rows0_v[i, sl] = rows0_v[i, sl] + rows1_v[i, sl]
                return carry

            jax.lax.fori_loop(c * CH, (c + 1) * CH, body_i, 0)
            pltpu.async_copy(rows0_v.at[r], out_hbm.at[pl.ds(base + c * CH, CH)],
                             s2.at[c])
        for c in range(NCH):
            r = pl.ds(c * CH, CH)
            pltpu.make_async_copy(rows0_v.at[r],
                                  out_hbm.at[pl.ds(base + c * CH, CH)],
                                  s2.at[c]).wait()

    return _sc_dispatch, _sc_combine


def kernel(x, kv_cache, Wg, bg, Wa, ba, W1, b1, W2, b2):
    B = x.shape[0]
    x2 = x.reshape(S, D)
    (pos0, pos1, w0, w1, te, act, first, nxt, do, bufp,
     nxt2, do2) = _routing(x2, Wg, bg)
    pos0w = pos0.reshape(_NW, _TPW)
    pos1w = pos1.reshape(_NW, _TPW)
    sc_dispatch, sc_combine = _sc_kernels()
    xs, rw = sc_dispatch(x2, pos0.reshape(_NW, _TPW // 16, 16),
                         pos1.reshape(_NW, _TPW // 16, 16), w0, w1)
    y = _ffn(xs, rw, te.reshape(NT), act.reshape(NT), first.reshape(NT),
             nxt.reshape(NT), do.reshape(NT), bufp.reshape(NT),
             nxt2.reshape(NT), do2.reshape(NT), Wa, ba, W1, b1, W2, b2)
    out = sc_combine(y, pos0w, pos1w)
    return out.astype(jnp.float16).reshape(B, S, D)
